# baseline (speedup 1.0000x reference)
import functools
import math

import jax
import jax.numpy as jnp
from jax import lax
from jax.experimental import pallas as pl
from jax.experimental.pallas import tpu as pltpu
from jax.experimental.pallas import tpu_sc as plsc

_NNUC = 2000
_NELEC = 48000
_NNODES = 50000
_EMB = 64
_DIN = 32
_E = 800000
_GCH = 512
_SCH = 256
_NC, _NS = 2, 16
_NW = _NC * _NS
_EPAD = 819200
_EPW = _EPAD // _NW
_EPT = _EPAD // _NS
_NPS = 24000
_NPSP = 24192
_RPT = _NPSP // _NS

_LOG_HALF = math.log(0.5)


def _ssp(x):
    return jnp.logaddexp(x, 0.0) + _LOG_HALF


def _tc_hx(nuc, elec, h_W, h_b):
    B = 400
    nb_nuc = _NNUC // B

    def body(n_ref, e_ref, w_ref, b_ref, o_ref):
        i = pl.program_id(0)

        @pl.when(i < nb_nuc)
        def _():
            o_ref[...] = n_ref[...]

        @pl.when(i >= nb_nuc)
        def _():
            o_ref[...] = (
                jnp.dot(e_ref[...], w_ref[...],
                        preferred_element_type=jnp.float32)
                + b_ref[...]
            )

    return pl.pallas_call(
        body,
        grid=(_NNODES // B,),
        in_specs=[
            pl.BlockSpec((B, _EMB), lambda i: (jnp.minimum(i, nb_nuc - 1), 0)),
            pl.BlockSpec((B, _EMB), lambda i: (jnp.maximum(i - nb_nuc, 0), 0)),
            pl.BlockSpec((_EMB, _EMB), lambda i: (0, 0)),
            pl.BlockSpec((1, _EMB), lambda i: (0, 0)),
        ],
        out_specs=pl.BlockSpec((B, _EMB), lambda i: (i, 0)),
        out_shape=jax.ShapeDtypeStruct((_NNODES, _EMB), jnp.float32),
    )(nuc, elec, h_W, h_b.reshape(1, _EMB))


def _tc_edge(dist, et2d, hs, W1c, b1c, W2v, b2n, b2s, b2a, Gv):
    B = 512
    last_dblk = (_E + B - 1) // B - 1

    def body(d_ref, t_ref, h_ref, w1_ref, b1_ref, w2_ref,
             bn_ref, bs_ref, ba_ref, g_ref, o_ref):
        et = t_ref[...]
        bn_m = et == 1
        bs_m = et == 3
        ba_m = et == 4
        mn = bn_m.astype(jnp.float32)
        ms = bs_m.astype(jnp.float32)
        ma = ba_m.astype(jnp.float32)
        h1 = _ssp(
            jnp.dot(d_ref[...], w1_ref[...], preferred_element_type=jnp.float32)
            + b1_ref[...]
        )
        h1m = jnp.concatenate(
            [jnp.where(bn_m, h1[:, 0:32], 0.0),
             jnp.where(bs_m, h1[:, 32:64], 0.0),
             jnp.where(ba_m, h1[:, 64:96], 0.0)], axis=1)
        we = jnp.dot(h1m, w2_ref[...], preferred_element_type=jnp.float32)
        we = we + mn * bn_ref[...] + ms * bs_ref[...] + ma * ba_ref[...]
        weh = we * h_ref[...]
        cat = jnp.concatenate(
            [jnp.where(bn_m, weh, 0.0),
             jnp.where(bs_m, weh, 0.0),
             jnp.where(ba_m, weh, 0.0)], axis=1)
        o_ref[...] = jnp.dot(cat, g_ref[...], preferred_element_type=jnp.float32)

    full = lambda i: (0, 0)
    return pl.pallas_call(
        body,
        grid=(_EPAD // B,),
        in_specs=[
            pl.BlockSpec((B, _DIN), lambda i: (jnp.minimum(i, last_dblk), 0)),
            pl.BlockSpec((B, 1), lambda i: (i, 0)),
            pl.BlockSpec((B, _EMB), lambda i: (i, 0)),
            pl.BlockSpec((_DIN, 96), full),
            pl.BlockSpec((1, 96), full),
            pl.BlockSpec((96, _EMB), full),
            pl.BlockSpec((1, _EMB), full),
            pl.BlockSpec((1, _EMB), full),
            pl.BlockSpec((1, _EMB), full),
            pl.BlockSpec((192, _EMB), full),
        ],
        out_specs=pl.BlockSpec((B, _EMB), lambda i: (i, 0)),
        out_shape=jax.ShapeDtypeStruct((_EPAD, _EMB), jnp.float32),
    )(dist, et2d, hs, W1c, b1c, W2v, b2n, b2s, b2a, Gv)


def _tc_out(elec, accf, bn, bs, ba):
    B = 96
    half = _NPS // B
    skip = (_NPSP - _NPS) // B

    def body(e_ref, u_ref, n_ref, s_ref, a_ref, o_ref):
        o_ref[...] = (e_ref[...] + u_ref[...]
                      + n_ref[...] + s_ref[...] + a_ref[...])

    full = lambda i: (0, 0)
    return pl.pallas_call(
        body,
        grid=(_NELEC // B,),
        in_specs=[
            pl.BlockSpec((B, _EMB), lambda i: (i, 0)),
            pl.BlockSpec((B, _EMB), lambda i: (jnp.where(i < half, i, i + skip), 0)),
            pl.BlockSpec((1, _EMB), full),
            pl.BlockSpec((1, _EMB), full),
            pl.BlockSpec((1, _EMB), full),
        ],
        out_specs=pl.BlockSpec((B, _EMB), lambda i: (i, 0)),
        out_shape=jax.ShapeDtypeStruct((_NELEC, _EMB), jnp.float32),
    )(elec, accf, bn.reshape(1, _EMB), bs.reshape(1, _EMB), ba.reshape(1, _EMB))


def _sc_gather(hx, snd2d):
    mesh = plsc.VectorSubcoreMesh(core_axis_name="c", subcore_axis_name="s")

    @functools.partial(
        pl.kernel, mesh=mesh,
        out_type=jax.ShapeDtypeStruct((_EPAD, _EMB), jnp.float32),
        compiler_params=pltpu.CompilerParams(use_tc_tiling_on_sc=False),
        scratch_types=[
            pltpu.VMEM((2, 4, 128), jnp.int32),
            pltpu.VMEM((2, _GCH, _EMB), jnp.float32),
            pltpu.SemaphoreType.DMA,
            pltpu.SemaphoreType.DMA,
            pltpu.SemaphoreType.DMA,
            pltpu.SemaphoreType.DMA,
        ],
    )
    def k(hx_hbm, snd_hbm, out_hbm, ibx, rows, gsa, gsb, wsa, wsb):
        wid = lax.axis_index("s") * _NC + lax.axis_index("c")
        rbase = wid * (_EPW // 128)
        ebase = wid * _EPW

        def body(p, carry):
            roff = rbase + p * 8
            eoff = ebase + p * (2 * _GCH)
            pltpu.sync_copy(snd_hbm.at[pl.ds(roff, 4)], ibx.at[0])
            ca = [pltpu.async_copy(hx_hbm.at[ibx.at[0].at[j]],
                                   rows.at[0].at[pl.ds(j * 128, 128)], gsa)
                  for j in range(4)]
            pltpu.sync_copy(snd_hbm.at[pl.ds(roff + 4, 4)], ibx.at[1])
            cb = [pltpu.async_copy(hx_hbm.at[ibx.at[1].at[j]],
                                   rows.at[1].at[pl.ds(j * 128, 128)], gsb)
                  for j in range(4)]
            for cp in ca:
                cp.wait()
            wa = pltpu.async_copy(rows.at[0], out_hbm.at[pl.ds(eoff, _GCH)], wsa)
            for cp in cb:
                cp.wait()
            wb = pltpu.async_copy(rows.at[1],
                                  out_hbm.at[pl.ds(eoff + _GCH, _GCH)], wsb)
            wa.wait()
            wb.wait()
            return carry

        lax.fori_loop(0, _EPW // (2 * _GCH), body, 0)

    return k(hx, snd2d)


def _sc_scatter(veh, rcv2d, zrows):
    mesh = plsc.VectorSubcoreMesh(core_axis_name="c", subcore_axis_name="s")

    @functools.partial(
        pl.kernel, mesh=mesh,
        out_type=jax.ShapeDtypeStruct((2 * _NPSP, _EMB), jnp.float32),
        compiler_params=pltpu.CompilerParams(use_tc_tiling_on_sc=False),
        scratch_types=[
            pltpu.VMEM((2, _SCH, _EMB), jnp.float32),
            pltpu.VMEM((2, 2, 128), jnp.int32),
            pltpu.VMEM((2, 2, 128), jnp.int32),
            pltpu.VMEM_SHARED((_NPSP, _EMB), jnp.float32),
            pltpu.SemaphoreType.DMA,
            pltpu.SemaphoreType.DMA,
        ],
    )
    def k(veh_hbm, rcv_hbm, z_hbm, out_hbm, vbufs, rbufs, ibufs, acc, vsa, vsb):
        c = lax.axis_index("c")
        s = lax.axis_index("s")
        nbase = _NNUC + c * _NPS
        pltpu.sync_copy(z_hbm, acc.at[pl.ds(s * _RPT, _RPT)])
        plsc.subcore_barrier()

        rtile = s * (_EPT // 128)
        etile = s * _EPT

        def body(p, carry):
            roff = rtile + p * 4
            eoff = etile + p * (2 * _SCH)
            vcp = []
            for b, vs in ((0, vsa), (1, vsb)):
                pltpu.sync_copy(rcv_hbm.at[pl.ds(roff + b * 2, 2)], rbufs.at[b])
                vcp.append(pltpu.async_copy(
                    veh_hbm.at[pl.ds(eoff + b * _SCH, _SCH)], vbufs.at[b], vs))
            for b in range(2):
                for j in range(2):
                    for t in range(8):
                        r = rbufs[b, j, pl.ds(t * 16, 16)]
                        rr = r - nbase
                        m = (rr >= 0) & (rr < _NPS)
                        ibufs[b, j, pl.ds(t * 16, 16)] = jnp.where(m, rr, _NPS)
            for b in range(2):
                vcp[b].wait()
                for j in range(2):
                    pltpu.sync_copy(vbufs.at[b].at[pl.ds(j * 128, 128)],
                                    acc.at[ibufs.at[b].at[j]], add=True)
            return carry

        lax.fori_loop(0, _EPT // (2 * _SCH), body, 0)
        plsc.subcore_barrier()
        pltpu.sync_copy(
            acc.at[pl.ds(s * _RPT, _RPT)],
            out_hbm.at[pl.ds(c * _NPSP + s * _RPT, _RPT)])

    return k(veh, rcv2d, zrows)


def kernel(nuc, elec, dist, edge_type, senders, receivers,
           w_same_W1, w_same_b1, w_same_W2, w_same_b2,
           w_anti_W1, w_anti_b1, w_anti_W2, w_anti_b2,
           w_n_W1, w_n_b1, w_n_W2, w_n_b2,
           g_same_W, g_same_b, g_anti_W, g_anti_b, g_n_W, g_n_b,
           h_W, h_b):
    pad = _EPAD - _E

    hx = _tc_hx(nuc, elec, h_W, h_b)

    snd = jnp.pad(senders.astype(jnp.int32), (0, pad)).reshape(_EPAD // 128, 128)
    hs = _sc_gather(hx, snd)

    et2d = jnp.pad(edge_type.astype(jnp.int32), (0, pad)).reshape(_EPAD, 1)
    W1c = jnp.concatenate([w_n_W1, w_same_W1, w_anti_W1], axis=1)
    b1c = jnp.concatenate([w_n_b1, w_same_b1, w_anti_b1]).reshape(1, 96)
    W2v = jnp.concatenate([w_n_W2, w_same_W2, w_anti_W2], axis=0)
    Gv = jnp.concatenate([g_n_W, g_same_W, g_anti_W], axis=0)
    veh = _tc_edge(dist, et2d, hs, W1c, b1c, W2v,
                   w_n_b2.reshape(1, _EMB), w_same_b2.reshape(1, _EMB),
                   w_anti_b2.reshape(1, _EMB), Gv)

    rcv = jnp.pad(receivers.astype(jnp.int32), (0, pad)).reshape(_EPAD // 128, 128)
    zrows = jnp.zeros((_RPT, _EMB), jnp.float32)
    accf = _sc_scatter(veh, rcv, zrows)

    return _tc_out(elec, accf, g_n_b, g_same_b, g_anti_b)


# speedup vs baseline: 1.0740x; 1.0740x over previous
import functools
import math

import jax
import jax.numpy as jnp
from jax import lax
from jax.experimental import pallas as pl
from jax.experimental.pallas import tpu as pltpu
from jax.experimental.pallas import tpu_sc as plsc

_NNUC = 2000
_NELEC = 48000
_NNODES = 50000
_EMB = 64
_DIN = 32
_E = 800000
_GCH = 512
_SCH = 256
_NC, _NS = 2, 16
_NW = _NC * _NS
_EPAD = 819200
_EPW = _EPAD // _NW
_EPT = _EPAD // _NS
_NPS = 24000
_NPSP = 24192
_RPT = _NPSP // _NS

_LOG_HALF = math.log(0.5)


def _ssp(x):
    return jnp.logaddexp(x, 0.0) + _LOG_HALF


def _tc_hx(nuc, elec, h_W, h_b):
    B = 400
    nb_nuc = _NNUC // B

    def body(n_ref, e_ref, w_ref, b_ref, o_ref):
        i = pl.program_id(0)

        @pl.when(i < nb_nuc)
        def _():
            o_ref[...] = n_ref[...]

        @pl.when(i >= nb_nuc)
        def _():
            o_ref[...] = (
                jnp.dot(e_ref[...], w_ref[...],
                        preferred_element_type=jnp.float32)
                + b_ref[...]
            )

    return pl.pallas_call(
        body,
        grid=(_NNODES // B,),
        in_specs=[
            pl.BlockSpec((B, _EMB), lambda i: (jnp.minimum(i, nb_nuc - 1), 0)),
            pl.BlockSpec((B, _EMB), lambda i: (jnp.maximum(i - nb_nuc, 0), 0)),
            pl.BlockSpec((_EMB, _EMB), lambda i: (0, 0)),
            pl.BlockSpec((1, _EMB), lambda i: (0, 0)),
        ],
        out_specs=pl.BlockSpec((B, _EMB), lambda i: (i, 0)),
        out_shape=jax.ShapeDtypeStruct((_NNODES, _EMB), jnp.float32),
    )(nuc, elec, h_W, h_b.reshape(1, _EMB))


def _tc_edge(dist, et2d, hs, W1c, b1c, W2v, b2n, b2s, b2a, Gv):
    B = 512

    def body(d_ref, t_ref, h_ref, w1_ref, b1_ref, w2_ref,
             bn_ref, bs_ref, ba_ref, g_ref, o_ref):
        et = t_ref[...]
        mn = (et == 1).astype(jnp.float32)
        ms = (et == 3).astype(jnp.float32)
        ma = (et == 4).astype(jnp.float32)
        h1 = _ssp(
            jnp.dot(d_ref[...], w1_ref[...], preferred_element_type=jnp.float32)
            + b1_ref[...]
        )
        h1m = jnp.concatenate(
            [h1[:, 0:32] * mn, h1[:, 32:64] * ms, h1[:, 64:96] * ma], axis=1)
        we = jnp.dot(h1m, w2_ref[...], preferred_element_type=jnp.float32)
        we = we + mn * bn_ref[...] + ms * bs_ref[...] + ma * ba_ref[...]
        weh = we * h_ref[...]
        cat = jnp.concatenate([weh * mn, weh * ms, weh * ma], axis=1)
        o_ref[...] = jnp.dot(cat, g_ref[...], preferred_element_type=jnp.float32)

    full = lambda i: (0, 0)
    return pl.pallas_call(
        body,
        grid=((_E + B - 1) // B,),
        in_specs=[
            pl.BlockSpec((B, _DIN), lambda i: (i, 0)),
            pl.BlockSpec((B, 1), lambda i: (i, 0)),
            pl.BlockSpec((B, _EMB), lambda i: (i, 0)),
            pl.BlockSpec((_DIN, 96), full),
            pl.BlockSpec((1, 96), full),
            pl.BlockSpec((96, _EMB), full),
            pl.BlockSpec((1, _EMB), full),
            pl.BlockSpec((1, _EMB), full),
            pl.BlockSpec((1, _EMB), full),
            pl.BlockSpec((192, _EMB), full),
        ],
        out_specs=pl.BlockSpec((B, _EMB), lambda i: (i, 0)),
        out_shape=jax.ShapeDtypeStruct((_EPAD, _EMB), jnp.float32),
    )(dist, et2d, hs, W1c, b1c, W2v, b2n, b2s, b2a, Gv)


def _tc_out(elec, accf, bn, bs, ba):
    B = 480

    def body(e_ref, u_ref, n_ref, s_ref, a_ref, o_ref):
        o_ref[...] = (e_ref[...] + u_ref[...]
                      + n_ref[...] + s_ref[...] + a_ref[...])

    full = lambda i: (0, 0)
    return pl.pallas_call(
        body,
        grid=(_NELEC // B,),
        in_specs=[
            pl.BlockSpec((B, _EMB), lambda i: (i, 0)),
            pl.BlockSpec((B, _EMB), lambda i: (i, 0)),
            pl.BlockSpec((1, _EMB), full),
            pl.BlockSpec((1, _EMB), full),
            pl.BlockSpec((1, _EMB), full),
        ],
        out_specs=pl.BlockSpec((B, _EMB), lambda i: (i, 0)),
        out_shape=jax.ShapeDtypeStruct((_NELEC, _EMB), jnp.float32),
    )(elec, accf, bn.reshape(1, _EMB), bs.reshape(1, _EMB), ba.reshape(1, _EMB))


def _sc_gather(hx, snd2d):
    mesh = plsc.VectorSubcoreMesh(core_axis_name="c", subcore_axis_name="s")

    @functools.partial(
        pl.kernel, mesh=mesh,
        out_type=jax.ShapeDtypeStruct((_EPAD, _EMB), jnp.float32),
        compiler_params=pltpu.CompilerParams(use_tc_tiling_on_sc=False),
        scratch_types=[
            pltpu.VMEM((2, 4, 128), jnp.int32),
            pltpu.VMEM((2, _GCH, _EMB), jnp.float32),
            pltpu.SemaphoreType.DMA,
            pltpu.SemaphoreType.DMA,
            pltpu.SemaphoreType.DMA,
            pltpu.SemaphoreType.DMA,
        ],
    )
    def k(hx_hbm, snd_hbm, out_hbm, ibx, rows, gsa, gsb, wsa, wsb):
        wid = lax.axis_index("s") * _NC + lax.axis_index("c")
        rbase = wid * (_EPW // 128)
        ebase = wid * _EPW

        def body(p, carry):
            roff = rbase + p * 8
            eoff = ebase + p * (2 * _GCH)
            pltpu.sync_copy(snd_hbm.at[pl.ds(roff, 4)], ibx.at[0])
            ca = [pltpu.async_copy(hx_hbm.at[ibx.at[0].at[j]],
                                   rows.at[0].at[pl.ds(j * 128, 128)], gsa)
                  for j in range(4)]
            pltpu.sync_copy(snd_hbm.at[pl.ds(roff + 4, 4)], ibx.at[1])
            cb = [pltpu.async_copy(hx_hbm.at[ibx.at[1].at[j]],
                                   rows.at[1].at[pl.ds(j * 128, 128)], gsb)
                  for j in range(4)]
            for cp in ca:
                cp.wait()
            wa = pltpu.async_copy(rows.at[0], out_hbm.at[pl.ds(eoff, _GCH)], wsa)
            for cp in cb:
                cp.wait()
            wb = pltpu.async_copy(rows.at[1],
                                  out_hbm.at[pl.ds(eoff + _GCH, _GCH)], wsb)
            wa.wait()
            wb.wait()
            return carry

        lax.fori_loop(0, _EPW // (2 * _GCH), body, 0)

    return k(hx, snd2d)


def _sc_scatter(veh, rcv2d, zrows):
    mesh = plsc.VectorSubcoreMesh(core_axis_name="c", subcore_axis_name="s")

    @functools.partial(
        pl.kernel, mesh=mesh,
        out_type=jax.ShapeDtypeStruct((_NELEC, _EMB), jnp.float32),
        compiler_params=pltpu.CompilerParams(use_tc_tiling_on_sc=False),
        scratch_types=[
            pltpu.VMEM((2, _SCH, _EMB), jnp.float32),
            pltpu.VMEM((2, 2, 128), jnp.int32),
            pltpu.VMEM((2, 2, 128), jnp.int32),
            pltpu.VMEM_SHARED((_NPSP, _EMB), jnp.float32),
            pltpu.SemaphoreType.DMA,
            pltpu.SemaphoreType.DMA,
        ],
    )
    def k(veh_hbm, rcv_hbm, z_hbm, out_hbm, vbufs, rbufs, ibufs, acc, vsa, vsb):
        c = lax.axis_index("c")
        s = lax.axis_index("s")
        nbase = _NNUC + c * _NPS
        pltpu.sync_copy(z_hbm, acc.at[pl.ds(s * _RPT, _RPT)])
        plsc.subcore_barrier()

        rtile = s * (_EPT // 128)
        etile = s * _EPT

        def body(p, carry):
            roff = rtile + p * 4
            eoff = etile + p * (2 * _SCH)
            vcp = []
            for b, vs in ((0, vsa), (1, vsb)):
                pltpu.sync_copy(rcv_hbm.at[pl.ds(roff + b * 2, 2)], rbufs.at[b])
                vcp.append(pltpu.async_copy(
                    veh_hbm.at[pl.ds(eoff + b * _SCH, _SCH)], vbufs.at[b], vs))
            for b in range(2):
                for j in range(2):
                    for t in range(8):
                        r = rbufs[b, j, pl.ds(t * 16, 16)]
                        rr = r - nbase
                        m = (rr >= 0) & (rr < _NPS)
                        ibufs[b, j, pl.ds(t * 16, 16)] = jnp.where(m, rr, _NPS)
            for b in range(2):
                vcp[b].wait()
                for j in range(2):
                    pltpu.sync_copy(vbufs.at[b].at[pl.ds(j * 128, 128)],
                                    acc.at[ibufs.at[b].at[j]], add=True)
            return carry

        lax.fori_loop(0, _EPT // (2 * _SCH), body, 0)
        plsc.subcore_barrier()
        rows = _NPS // _NS
        pltpu.sync_copy(
            acc.at[pl.ds(s * rows, rows)],
            out_hbm.at[pl.ds(c * _NPS + s * rows, rows)])

    return k(veh, rcv2d, zrows)


def kernel(nuc, elec, dist, edge_type, senders, receivers,
           w_same_W1, w_same_b1, w_same_W2, w_same_b2,
           w_anti_W1, w_anti_b1, w_anti_W2, w_anti_b2,
           w_n_W1, w_n_b1, w_n_W2, w_n_b2,
           g_same_W, g_same_b, g_anti_W, g_anti_b, g_n_W, g_n_b,
           h_W, h_b):
    pad = _EPAD - _E

    hx = _tc_hx(nuc, elec, h_W, h_b)

    snd = jnp.pad(senders.astype(jnp.int32), (0, pad)).reshape(_EPAD // 128, 128)
    hs = _sc_gather(hx, snd)

    et2d = edge_type.astype(jnp.int32).reshape(_E, 1)
    W1c = jnp.concatenate([w_n_W1, w_same_W1, w_anti_W1], axis=1)
    b1c = jnp.concatenate([w_n_b1, w_same_b1, w_anti_b1]).reshape(1, 96)
    W2v = jnp.concatenate([w_n_W2, w_same_W2, w_anti_W2], axis=0)
    Gv = jnp.concatenate([g_n_W, g_same_W, g_anti_W], axis=0)
    veh = _tc_edge(dist, et2d, hs, W1c, b1c, W2v,
                   w_n_b2.reshape(1, _EMB), w_same_b2.reshape(1, _EMB),
                   w_anti_b2.reshape(1, _EMB), Gv)

    rcv = jnp.pad(receivers.astype(jnp.int32), (0, pad)).reshape(_EPAD // 128, 128)
    zrows = jnp.zeros((_RPT, _EMB), jnp.float32)
    accf = _sc_scatter(veh, rcv, zrows)

    return _tc_out(elec, accf, g_n_b, g_same_b, g_anti_b)


# speedup vs baseline: 1.1454x; 1.0665x over previous
import functools
import math

import jax
import jax.numpy as jnp
from jax import lax
from jax.experimental import pallas as pl
from jax.experimental.pallas import tpu as pltpu
from jax.experimental.pallas import tpu_sc as plsc

_NNUC = 2000
_NELEC = 48000
_NNODES = 50000
_EMB = 64
_DIN = 32
_E = 800000
_GCH = 512
_SCH = 256
_NC, _NS = 2, 16
_NW = _NC * _NS
_EPAD = 819200
_EPW = _EPAD // _NW
_EPT = _EPAD // _NS
_NPS = 24000
_NPSP = 24192
_RPT = _NPSP // _NS

_LOG_HALF = math.log(0.5)


def _ssp(x):
    return jnp.logaddexp(x, 0.0) + _LOG_HALF


def _tc_hx(nuc, elec, h_W, h_b):
    B = 400
    nb_nuc = _NNUC // B

    def body(n_ref, e_ref, w_ref, b_ref, o_ref):
        i = pl.program_id(0)

        @pl.when(i < nb_nuc)
        def _():
            o_ref[...] = n_ref[...].astype(jnp.bfloat16)

        @pl.when(i >= nb_nuc)
        def _():
            o_ref[...] = (
                jnp.dot(e_ref[...], w_ref[...],
                        preferred_element_type=jnp.float32)
                + b_ref[...]
            ).astype(jnp.bfloat16)

    return pl.pallas_call(
        body,
        grid=(_NNODES // B,),
        in_specs=[
            pl.BlockSpec((B, _EMB), lambda i: (jnp.minimum(i, nb_nuc - 1), 0)),
            pl.BlockSpec((B, _EMB), lambda i: (jnp.maximum(i - nb_nuc, 0), 0)),
            pl.BlockSpec((_EMB, _EMB), lambda i: (0, 0)),
            pl.BlockSpec((1, _EMB), lambda i: (0, 0)),
        ],
        out_specs=pl.BlockSpec((B, _EMB), lambda i: (i, 0)),
        out_shape=jax.ShapeDtypeStruct((_NNODES, _EMB), jnp.bfloat16),
    )(nuc, elec, h_W, h_b.reshape(1, _EMB))


def _tc_edge(dist, et2d, hs, W1c, b1c, W2v, b2n, b2s, b2a, Gv):
    B = 512

    def body(d_ref, t_ref, h_ref, w1_ref, b1_ref, w2_ref,
             bn_ref, bs_ref, ba_ref, g_ref, o_ref):
        et = t_ref[...]
        mn = (et == 1).astype(jnp.float32)
        ms = (et == 3).astype(jnp.float32)
        ma = (et == 4).astype(jnp.float32)
        h1 = _ssp(
            jnp.dot(d_ref[...], w1_ref[...], preferred_element_type=jnp.float32)
            + b1_ref[...]
        )
        h1m = jnp.concatenate(
            [h1[:, 0:32] * mn, h1[:, 32:64] * ms, h1[:, 64:96] * ma], axis=1)
        we = jnp.dot(h1m, w2_ref[...], preferred_element_type=jnp.float32)
        we = we + mn * bn_ref[...] + ms * bs_ref[...] + ma * ba_ref[...]
        weh = we * h_ref[...].astype(jnp.float32)
        cat = jnp.concatenate([weh * mn, weh * ms, weh * ma], axis=1)
        o_ref[...] = jnp.dot(cat, g_ref[...], preferred_element_type=jnp.float32)

    full = lambda i: (0, 0)
    return pl.pallas_call(
        body,
        grid=((_E + B - 1) // B,),
        in_specs=[
            pl.BlockSpec((B, _DIN), lambda i: (i, 0)),
            pl.BlockSpec((B, 1), lambda i: (i, 0)),
            pl.BlockSpec((B, _EMB), lambda i: (i, 0)),
            pl.BlockSpec((_DIN, 96), full),
            pl.BlockSpec((1, 96), full),
            pl.BlockSpec((96, _EMB), full),
            pl.BlockSpec((1, _EMB), full),
            pl.BlockSpec((1, _EMB), full),
            pl.BlockSpec((1, _EMB), full),
            pl.BlockSpec((192, _EMB), full),
        ],
        out_specs=pl.BlockSpec((B, _EMB), lambda i: (i, 0)),
        out_shape=jax.ShapeDtypeStruct((_EPAD, _EMB), jnp.float32),
    )(dist, et2d, hs, W1c, b1c, W2v, b2n, b2s, b2a, Gv)


def _tc_out(elec, accf, bn, bs, ba):
    B = 480

    def body(e_ref, u_ref, n_ref, s_ref, a_ref, o_ref):
        o_ref[...] = (e_ref[...] + u_ref[...]
                      + n_ref[...] + s_ref[...] + a_ref[...])

    full = lambda i: (0, 0)
    return pl.pallas_call(
        body,
        grid=(_NELEC // B,),
        in_specs=[
            pl.BlockSpec((B, _EMB), lambda i: (i, 0)),
            pl.BlockSpec((B, _EMB), lambda i: (i, 0)),
            pl.BlockSpec((1, _EMB), full),
            pl.BlockSpec((1, _EMB), full),
            pl.BlockSpec((1, _EMB), full),
        ],
        out_specs=pl.BlockSpec((B, _EMB), lambda i: (i, 0)),
        out_shape=jax.ShapeDtypeStruct((_NELEC, _EMB), jnp.float32),
    )(elec, accf, bn.reshape(1, _EMB), bs.reshape(1, _EMB), ba.reshape(1, _EMB))


def _sc_gather(hx, snd2d):
    mesh = plsc.VectorSubcoreMesh(core_axis_name="c", subcore_axis_name="s")

    @functools.partial(
        pl.kernel, mesh=mesh,
        out_type=jax.ShapeDtypeStruct((_EPAD, _EMB), jnp.bfloat16),
        compiler_params=pltpu.CompilerParams(use_tc_tiling_on_sc=False),
        scratch_types=[
            pltpu.VMEM((2, 4, 128), jnp.int32),
            pltpu.VMEM((2, _GCH, _EMB), jnp.bfloat16),
            pltpu.SemaphoreType.DMA,
            pltpu.SemaphoreType.DMA,
            pltpu.SemaphoreType.DMA,
            pltpu.SemaphoreType.DMA,
        ],
    )
    def k(hx_hbm, snd_hbm, out_hbm, ibx, rows, gsa, gsb, wsa, wsb):
        wid = lax.axis_index("s") * _NC + lax.axis_index("c")
        rbase = wid * (_EPW // 128)
        ebase = wid * _EPW

        def body(p, carry):
            roff = rbase + p * 8
            eoff = ebase + p * (2 * _GCH)
            pltpu.sync_copy(snd_hbm.at[pl.ds(roff, 4)], ibx.at[0])
            ca = [pltpu.async_copy(hx_hbm.at[ibx.at[0].at[j]],
                                   rows.at[0].at[pl.ds(j * 128, 128)], gsa)
                  for j in range(4)]
            pltpu.sync_copy(snd_hbm.at[pl.ds(roff + 4, 4)], ibx.at[1])
            cb = [pltpu.async_copy(hx_hbm.at[ibx.at[1].at[j]],
                                   rows.at[1].at[pl.ds(j * 128, 128)], gsb)
                  for j in range(4)]
            for cp in ca:
                cp.wait()
            wa = pltpu.async_copy(rows.at[0], out_hbm.at[pl.ds(eoff, _GCH)], wsa)
            for cp in cb:
                cp.wait()
            wb = pltpu.async_copy(rows.at[1],
                                  out_hbm.at[pl.ds(eoff + _GCH, _GCH)], wsb)
            wa.wait()
            wb.wait()
            return carry

        lax.fori_loop(0, _EPW // (2 * _GCH), body, 0)

    return k(hx, snd2d)


def _sc_scatter(veh, rcv2d, zrows):
    mesh = plsc.VectorSubcoreMesh(core_axis_name="c", subcore_axis_name="s")

    @functools.partial(
        pl.kernel, mesh=mesh,
        out_type=jax.ShapeDtypeStruct((_NELEC, _EMB), jnp.float32),
        compiler_params=pltpu.CompilerParams(use_tc_tiling_on_sc=False),
        scratch_types=[
            pltpu.VMEM((2, _SCH, _EMB), jnp.float32),
            pltpu.VMEM((2, 2, 128), jnp.int32),
            pltpu.VMEM((2, 2, 128), jnp.int32),
            pltpu.VMEM_SHARED((_NPSP, _EMB), jnp.float32),
            pltpu.SemaphoreType.DMA,
            pltpu.SemaphoreType.DMA,
        ],
    )
    def k(veh_hbm, rcv_hbm, z_hbm, out_hbm, vbufs, rbufs, ibufs, acc, vsa, vsb):
        c = lax.axis_index("c")
        s = lax.axis_index("s")
        nbase = _NNUC + c * _NPS
        pltpu.sync_copy(z_hbm, acc.at[pl.ds(s * _RPT, _RPT)])
        plsc.subcore_barrier()

        rtile = s * (_EPT // 128)
        etile = s * _EPT

        def body(p, carry):
            roff = rtile + p * 4
            eoff = etile + p * (2 * _SCH)
            vcp = []
            for b, vs in ((0, vsa), (1, vsb)):
                pltpu.sync_copy(rcv_hbm.at[pl.ds(roff + b * 2, 2)], rbufs.at[b])
                vcp.append(pltpu.async_copy(
                    veh_hbm.at[pl.ds(eoff + b * _SCH, _SCH)], vbufs.at[b], vs))
            for b in range(2):
                for j in range(2):
                    for t in range(8):
                        r = rbufs[b, j, pl.ds(t * 16, 16)]
                        rr = r - nbase
                        m = (rr >= 0) & (rr < _NPS)
                        dummy = _NPS + jnp.bitwise_and(r, 127)
                        ibufs[b, j, pl.ds(t * 16, 16)] = jnp.where(m, rr, dummy)
            for b in range(2):
                vcp[b].wait()
                for j in range(2):
                    pltpu.sync_copy(vbufs.at[b].at[pl.ds(j * 128, 128)],
                                    acc.at[ibufs.at[b].at[j]], add=True)
            return carry

        lax.fori_loop(0, _EPT // (2 * _SCH), body, 0)
        plsc.subcore_barrier()
        rows = _NPS // _NS
        pltpu.sync_copy(
            acc.at[pl.ds(s * rows, rows)],
            out_hbm.at[pl.ds(c * _NPS + s * rows, rows)])

    return k(veh, rcv2d, zrows)


def kernel(nuc, elec, dist, edge_type, senders, receivers,
           w_same_W1, w_same_b1, w_same_W2, w_same_b2,
           w_anti_W1, w_anti_b1, w_anti_W2, w_anti_b2,
           w_n_W1, w_n_b1, w_n_W2, w_n_b2,
           g_same_W, g_same_b, g_anti_W, g_anti_b, g_n_W, g_n_b,
           h_W, h_b):
    pad = _EPAD - _E

    hx = _tc_hx(nuc, elec, h_W, h_b)

    snd = jnp.pad(senders.astype(jnp.int32), (0, pad)).reshape(_EPAD // 128, 128)
    hs = _sc_gather(hx, snd)

    et2d = edge_type.astype(jnp.int32).reshape(_E, 1)
    W1c = jnp.concatenate([w_n_W1, w_same_W1, w_anti_W1], axis=1)
    b1c = jnp.concatenate([w_n_b1, w_same_b1, w_anti_b1]).reshape(1, 96)
    W2v = jnp.concatenate([w_n_W2, w_same_W2, w_anti_W2], axis=0)
    Gv = jnp.concatenate([g_n_W, g_same_W, g_anti_W], axis=0)
    veh = _tc_edge(dist, et2d, hs, W1c, b1c, W2v,
                   w_n_b2.reshape(1, _EMB), w_same_b2.reshape(1, _EMB),
                   w_anti_b2.reshape(1, _EMB), Gv)

    rcv = jnp.pad(receivers.astype(jnp.int32), (0, pad)).reshape(_EPAD // 128, 128)
    zrows = jnp.zeros((_RPT, _EMB), jnp.float32)
    accf = _sc_scatter(veh, rcv, zrows)

    return _tc_out(elec, accf, g_n_b, g_same_b, g_anti_b)


# speedup vs baseline: 1.3887x; 1.2125x over previous
import functools
import math

import jax
import jax.numpy as jnp
from jax import lax
from jax.experimental import pallas as pl
from jax.experimental.pallas import tpu as pltpu
from jax.experimental.pallas import tpu_sc as plsc

_NNUC = 2000
_NELEC = 48000
_NNODES = 50000
_EMB = 64
_DIN = 32
_E = 800000
_GCH = 512
_SCH = 256
_NC, _NS = 2, 16
_NW = _NC * _NS
_EPAD = 819200
_EPW = _EPAD // _NW
_EPT = _EPAD // _NS
_NPS = 24000
_NPSP = 24192
_RPT = _NPSP // _NS

_LOG_HALF = math.log(0.5)


def _ssp(x):
    return jnp.logaddexp(x, 0.0) + _LOG_HALF


def _tc_hx(nuc, elec, h_W, h_b):
    B = 400
    nb_nuc = _NNUC // B

    def body(n_ref, e_ref, w_ref, b_ref, o_ref):
        i = pl.program_id(0)

        @pl.when(i < nb_nuc)
        def _():
            o_ref[...] = n_ref[...].astype(jnp.bfloat16)

        @pl.when(i >= nb_nuc)
        def _():
            o_ref[...] = (
                jnp.dot(e_ref[...], w_ref[...],
                        preferred_element_type=jnp.float32)
                + b_ref[...]
            ).astype(jnp.bfloat16)

    return pl.pallas_call(
        body,
        grid=(_NNODES // B,),
        in_specs=[
            pl.BlockSpec((B, _EMB), lambda i: (jnp.minimum(i, nb_nuc - 1), 0)),
            pl.BlockSpec((B, _EMB), lambda i: (jnp.maximum(i - nb_nuc, 0), 0)),
            pl.BlockSpec((_EMB, _EMB), lambda i: (0, 0)),
            pl.BlockSpec((1, _EMB), lambda i: (0, 0)),
        ],
        out_specs=pl.BlockSpec((B, _EMB), lambda i: (i, 0)),
        out_shape=jax.ShapeDtypeStruct((_NNODES, _EMB), jnp.bfloat16),
    )(nuc, elec, h_W, h_b.reshape(1, _EMB))


def _tc_edge(dpk, hs, W1c, b1c, W2v, b2n, b2s, b2a, Gv):
    B = 1024
    H = B // 2

    def body(d_ref, h_ref, w1_ref, b1_ref, w2_ref,
             bn_ref, bs_ref, ba_ref, g_ref, o_ref):
        t = d_ref[:, _DIN:_DIN + 1].astype(jnp.float32)
        mn = (t == 1.0).astype(jnp.float32)
        ms = (t == 3.0).astype(jnp.float32)
        ma = (t == 4.0).astype(jnp.float32)
        x1 = (
            jnp.dot(d_ref[:, 0:_DIN], w1_ref[...],
                    preferred_element_type=jnp.float32)
            + b1_ref[...]
        )
        xs = x1[:, 0:32] * mn + x1[:, 32:64] * ms + x1[:, 64:96] * ma
        h1 = _ssp(xs)
        h1m = jnp.concatenate([h1 * mn, h1 * ms, h1 * ma], axis=1)
        we = jnp.dot(h1m, w2_ref[...], preferred_element_type=jnp.float32)
        we = we + mn * bn_ref[...] + ms * bs_ref[...] + ma * ba_ref[...]
        weh = we * h_ref[...].astype(jnp.float32)
        cat = jnp.concatenate([weh * mn, weh * ms, weh * ma], axis=1)
        o_ref[...] = jnp.concatenate(
            [jnp.dot(cat[0:H], g_ref[...], preferred_element_type=jnp.float32),
             jnp.dot(cat[H:B], g_ref[...], preferred_element_type=jnp.float32)],
            axis=1)

    full = lambda i: (0, 0)
    return pl.pallas_call(
        body,
        grid=(_EPAD // B,),
        in_specs=[
            pl.BlockSpec((B, _DIN + 1), lambda i: (i, 0)),
            pl.BlockSpec((B, _EMB), lambda i: (i, 0)),
            pl.BlockSpec((_DIN, 96), full),
            pl.BlockSpec((1, 96), full),
            pl.BlockSpec((96, _EMB), full),
            pl.BlockSpec((1, _EMB), full),
            pl.BlockSpec((1, _EMB), full),
            pl.BlockSpec((1, _EMB), full),
            pl.BlockSpec((192, _EMB), full),
        ],
        out_specs=pl.BlockSpec((H, 2 * _EMB), lambda i: (i, 0)),
        out_shape=jax.ShapeDtypeStruct((_EPAD // 2, 2 * _EMB), jnp.float32),
    )(dpk, hs, W1c, b1c, W2v, b2n, b2s, b2a, Gv)


def _tc_out(elec, accf, bn, bs, ba):
    B = 480

    def body(e_ref, u_ref, n_ref, s_ref, a_ref, o_ref):
        o_ref[...] = (e_ref[...] + u_ref[...]
                      + n_ref[...] + s_ref[...] + a_ref[...])

    full = lambda i: (0, 0)
    return pl.pallas_call(
        body,
        grid=(_NELEC // B,),
        in_specs=[
            pl.BlockSpec((B, _EMB), lambda i: (i, 0)),
            pl.BlockSpec((B, _EMB), lambda i: (i, 0)),
            pl.BlockSpec((1, _EMB), full),
            pl.BlockSpec((1, _EMB), full),
            pl.BlockSpec((1, _EMB), full),
        ],
        out_specs=pl.BlockSpec((B, _EMB), lambda i: (i, 0)),
        out_shape=jax.ShapeDtypeStruct((_NELEC, _EMB), jnp.float32),
    )(elec, accf, bn.reshape(1, _EMB), bs.reshape(1, _EMB), ba.reshape(1, _EMB))


def _sc_gather(hx, snd2d):
    mesh = plsc.VectorSubcoreMesh(core_axis_name="c", subcore_axis_name="s")

    @functools.partial(
        pl.kernel, mesh=mesh,
        out_type=jax.ShapeDtypeStruct((_EPAD, _EMB), jnp.bfloat16),
        compiler_params=pltpu.CompilerParams(use_tc_tiling_on_sc=False),
        scratch_types=[
            pltpu.VMEM((2, 4, 128), jnp.int32),
            pltpu.VMEM((2, _GCH, _EMB), jnp.bfloat16),
            pltpu.SemaphoreType.DMA,
            pltpu.SemaphoreType.DMA,
            pltpu.SemaphoreType.DMA,
            pltpu.SemaphoreType.DMA,
        ],
    )
    def k(hx_hbm, snd_hbm, out_hbm, ibx, rows, gsa, gsb, wsa, wsb):
        wid = lax.axis_index("s") * _NC + lax.axis_index("c")
        rbase = wid * (_EPW // 128)
        ebase = wid * _EPW

        def body(p, carry):
            roff = rbase + p * 8
            eoff = ebase + p * (2 * _GCH)
            pltpu.sync_copy(snd_hbm.at[pl.ds(roff, 4)], ibx.at[0])
            ca = [pltpu.async_copy(hx_hbm.at[ibx.at[0].at[j]],
                                   rows.at[0].at[pl.ds(j * 128, 128)], gsa)
                  for j in range(4)]
            pltpu.sync_copy(snd_hbm.at[pl.ds(roff + 4, 4)], ibx.at[1])
            cb = [pltpu.async_copy(hx_hbm.at[ibx.at[1].at[j]],
                                   rows.at[1].at[pl.ds(j * 128, 128)], gsb)
                  for j in range(4)]
            for cp in ca:
                cp.wait()
            wa = pltpu.async_copy(rows.at[0], out_hbm.at[pl.ds(eoff, _GCH)], wsa)
            for cp in cb:
                cp.wait()
            wb = pltpu.async_copy(rows.at[1],
                                  out_hbm.at[pl.ds(eoff + _GCH, _GCH)], wsb)
            wa.wait()
            wb.wait()
            return carry

        lax.fori_loop(0, _EPW // (2 * _GCH), body, 0)

    return k(hx, snd2d)


def _sc_scatter(veh, rcv2d, zrows):
    mesh = plsc.VectorSubcoreMesh(core_axis_name="c", subcore_axis_name="s")

    @functools.partial(
        pl.kernel, mesh=mesh,
        out_type=jax.ShapeDtypeStruct((_NELEC, _EMB), jnp.float32),
        compiler_params=pltpu.CompilerParams(use_tc_tiling_on_sc=False),
        scratch_types=[
            pltpu.VMEM((2, _SCH, _EMB), jnp.float32),
            pltpu.VMEM((2, 2, 128), jnp.int32),
            pltpu.VMEM((2, 2, 128), jnp.int32),
            pltpu.VMEM_SHARED((_NPSP, _EMB), jnp.float32),
            pltpu.SemaphoreType.DMA,
            pltpu.SemaphoreType.DMA,
        ],
    )
    def k(veh_hbm, rcv_hbm, z_hbm, out_hbm, vbufs, rbufs, ibufs, acc, vsa, vsb):
        c = lax.axis_index("c")
        s = lax.axis_index("s")
        nbase = _NNUC + c * _NPS
        pltpu.sync_copy(z_hbm, acc.at[pl.ds(s * _RPT, _RPT)])
        plsc.subcore_barrier()

        rtile = s * (_EPT // 128)
        etile = s * _EPT

        def body(p, carry):
            roff = rtile + p * 4
            eoff = etile + p * (2 * _SCH)
            vcp = []
            for b, vs in ((0, vsa), (1, vsb)):
                pltpu.sync_copy(rcv_hbm.at[pl.ds(roff + b * 2, 2)], rbufs.at[b])
                vcp.append(pltpu.async_copy(
                    veh_hbm.at[pl.ds(eoff + b * _SCH, _SCH)], vbufs.at[b], vs))
            for b in range(2):
                for j in range(2):
                    for t in range(8):
                        r = rbufs[b, j, pl.ds(t * 16, 16)]
                        rr = r - nbase
                        m = (rr >= 0) & (rr < _NPS)
                        dummy = _NPS + jnp.bitwise_and(r, 127)
                        ibufs[b, j, pl.ds(t * 16, 16)] = jnp.where(m, rr, dummy)
            for b in range(2):
                vcp[b].wait()
                for j in range(2):
                    pltpu.sync_copy(vbufs.at[b].at[pl.ds(j * 128, 128)],
                                    acc.at[ibufs.at[b].at[j]], add=True)
            return carry

        lax.fori_loop(0, _EPT // (2 * _SCH), body, 0)
        plsc.subcore_barrier()
        rows = _NPS // _NS
        pltpu.sync_copy(
            acc.at[pl.ds(s * rows, rows)],
            out_hbm.at[pl.ds(c * _NPS + s * rows, rows)])

    return k(veh, rcv2d, zrows)


def kernel(nuc, elec, dist, edge_type, senders, receivers,
           w_same_W1, w_same_b1, w_same_W2, w_same_b2,
           w_anti_W1, w_anti_b1, w_anti_W2, w_anti_b2,
           w_n_W1, w_n_b1, w_n_W2, w_n_b2,
           g_same_W, g_same_b, g_anti_W, g_anti_b, g_n_W, g_n_b,
           h_W, h_b):
    pad = _EPAD - _E

    hx = _tc_hx(nuc, elec, h_W, h_b)

    snd = jnp.pad(senders.astype(jnp.int32), (0, pad)).reshape(_EPAD // 128, 128)
    hs = _sc_gather(hx, snd)

    dpk = jnp.pad(
        jnp.concatenate(
            [dist.astype(jnp.bfloat16),
             edge_type.astype(jnp.bfloat16).reshape(_E, 1)], axis=1),
        ((0, pad), (0, 0)))
    W1c = jnp.concatenate(
        [w_n_W1, w_same_W1, w_anti_W1], axis=1).astype(jnp.bfloat16)
    b1c = jnp.concatenate([w_n_b1, w_same_b1, w_anti_b1]).reshape(1, 96)
    W2v = jnp.concatenate([w_n_W2, w_same_W2, w_anti_W2], axis=0)
    Gv = jnp.concatenate([g_n_W, g_same_W, g_anti_W], axis=0)
    veh = _tc_edge(dpk, hs, W1c, b1c, W2v,
                   w_n_b2.reshape(1, _EMB), w_same_b2.reshape(1, _EMB),
                   w_anti_b2.reshape(1, _EMB), Gv).reshape(_EPAD, _EMB)

    rcv = (jnp.pad(receivers.astype(jnp.int32), (0, pad))
           .reshape(-1, 2, 512).transpose(0, 2, 1)
           .reshape(_EPAD // 128, 128))
    zrows = jnp.zeros((_RPT, _EMB), jnp.float32)
    accf = _sc_scatter(veh, rcv, zrows)

    return _tc_out(elec, accf, g_n_b, g_same_b, g_anti_b)
